# Initial kernel scaffold; baseline (speedup 1.0000x reference)
import jax
import jax.numpy as jnp
from jax.experimental import pallas as pl


def kernel(input_ids, wte, wpe):
    raise NotImplementedError("write your pallas kernel here")


# speedup vs baseline: 1.0105x; 1.0105x over previous
import functools

import jax
import jax.numpy as jnp
from jax import lax
from jax.experimental import pallas as pl
from jax.experimental.pallas import tpu as pltpu
from jax.experimental.pallas import tpu_sc as plsc

VOCAB = 50257
D = 768
B = 4
S = 2048
N = S * B
NC = 2
NS = 16
NW = NC * NS
RPW = N // NW
SPW = S // NW
NCH = 4
C = RPW // NCH
CS = C // B
LANES = 16
NDB = D // LANES


def _sc_embed(idx3, wte, wpe):
    mesh = plsc.VectorSubcoreMesh(core_axis_name="c", subcore_axis_name="s")

    @functools.partial(
        pl.kernel,
        mesh=mesh,
        out_type=jax.ShapeDtypeStruct((N, D), jnp.float32),
        scratch_types=[
            pltpu.VMEM((NCH, C), jnp.int32),
            pltpu.VMEM((C, D), jnp.float32),
            pltpu.VMEM((SPW, D), jnp.float32),
            pltpu.SemaphoreType.DMA,
        ],
    )
    def k(idx_hbm, wte_hbm, wpe_hbm, out_hbm, idx_v, rows_v, wpe_v, sem):
        wid = lax.axis_index("s") * NC + lax.axis_index("c")
        base = wid * RPW
        s0 = wid * SPW
        pltpu.sync_copy(idx_hbm.at[wid], idx_v)
        pltpu.sync_copy(wpe_hbm.at[pl.ds(s0, SPW)], wpe_v)
        for c in range(NCH):
            pltpu.async_copy(wte_hbm.at[idx_v.at[c]], rows_v, sem).wait()

            def add_body(i, carry):
                off = i * LANES
                for sl in range(CS):
                    w = wpe_v[c * CS + sl, pl.ds(off, LANES)]
                    for b in range(B):
                        r = sl * B + b
                        rows_v[r, pl.ds(off, LANES)] = (
                            rows_v[r, pl.ds(off, LANES)] + w
                        )
                return carry

            lax.fori_loop(0, NDB, add_body, 0)
            pltpu.sync_copy(rows_v, out_hbm.at[pl.ds(base + c * C, C)])

    return k(idx3, wte, wpe)


def kernel(input_ids, wte, wpe):
    idx3 = jnp.transpose(input_ids).reshape(NW, NCH, C)
    flat = _sc_embed(idx3, wte, wpe)
    hidden = flat.reshape(S, B, D)
    return (hidden, input_ids)


# speedup vs baseline: 1.7464x; 1.7282x over previous
import functools

import jax
import jax.numpy as jnp
from jax import lax
from jax.experimental import pallas as pl
from jax.experimental.pallas import tpu as pltpu
from jax.experimental.pallas import tpu_sc as plsc

VOCAB = 50257
D = 768
B = 4
S = 2048
N = S * B
NC = 2
NS = 16
NW = NC * NS
RPW = N // NW
SPW = S // NW
NCH = 8
C = RPW // NCH
CS = C // B
LANES = 16
NDB = D // LANES
NBUF = 3


def _sc_embed(idx3, wte, wpe):
    mesh = plsc.VectorSubcoreMesh(core_axis_name="c", subcore_axis_name="s")

    @functools.partial(
        pl.kernel,
        mesh=mesh,
        out_type=jax.ShapeDtypeStruct((S, B, D), jnp.float32),
        scratch_types=[
            pltpu.VMEM((NCH, C), jnp.int32),
            pltpu.VMEM((C, D), jnp.float32),
            pltpu.VMEM((C, D), jnp.float32),
            pltpu.VMEM((C, D), jnp.float32),
            pltpu.VMEM((SPW, D), jnp.float32),
            pltpu.SemaphoreType.DMA,
            pltpu.SemaphoreType.DMA,
            pltpu.SemaphoreType.DMA,
            pltpu.SemaphoreType.DMA,
            pltpu.SemaphoreType.DMA,
            pltpu.SemaphoreType.DMA,
            pltpu.SemaphoreType.DMA,
        ],
    )
    def k(idx_hbm, wte_hbm, wpe_hbm, out_hbm,
          idx_v, r0, r1, r2, wpe_v, g0, g1, g2, w0, w1, w2, pe_sem):
        bufs = (r0, r1, r2)
        gsems = (g0, g1, g2)
        wsems = (w0, w1, w2)
        wid = lax.axis_index("s") * NC + lax.axis_index("c")
        s0 = wid * SPW
        pltpu.sync_copy(idx_hbm.at[wid], idx_v)
        pe_desc = pltpu.async_copy(wpe_hbm.at[pl.ds(s0, SPW)], wpe_v, pe_sem)
        gdescs = [None] * NBUF
        wdescs = [None] * NBUF
        for j in range(NBUF):
            gdescs[j] = pltpu.async_copy(
                wte_hbm.at[idx_v.at[j]], bufs[j], gsems[j])
        pe_desc.wait()
        for j in range(NCH):
            b = j % NBUF
            if j >= 2 and j + 1 < NCH:
                nb = (j + 1) % NBUF
                for d in wdescs[nb]:
                    d.wait()
                gdescs[nb] = pltpu.async_copy(
                    wte_hbm.at[idx_v.at[j + 1]], bufs[nb], gsems[nb])
            gdescs[b].wait()
            buf = bufs[b]

            def add_body(i, carry, buf=buf, j=j):
                off = i * LANES
                for sl in range(CS):
                    w = wpe_v[j * CS + sl, pl.ds(off, LANES)]
                    for bb in range(B):
                        r = sl * B + bb
                        buf[r, pl.ds(off, LANES)] = (
                            buf[r, pl.ds(off, LANES)] + w
                        )
                return carry

            lax.fori_loop(0, NDB, add_body, 0)
            ds = []
            for sl in range(CS):
                ds.append(pltpu.async_copy(
                    buf.at[pl.ds(sl * B, B)],
                    out_hbm.at[s0 + j * CS + sl],
                    wsems[b]))
            wdescs[b] = ds
        for j in range(NCH - NBUF, NCH):
            for d in wdescs[j % NBUF]:
                d.wait()

    return k(idx3, wte, wpe)


def kernel(input_ids, wte, wpe):
    idx3 = jnp.transpose(input_ids).reshape(NW, NCH, C)
    hidden = _sc_embed(idx3, wte, wpe)
    return (hidden, input_ids)


# speedup vs baseline: 1.7670x; 1.0118x over previous
import functools

import jax
import jax.numpy as jnp
from jax import lax
from jax.experimental import pallas as pl
from jax.experimental.pallas import tpu as pltpu
from jax.experimental.pallas import tpu_sc as plsc

VOCAB = 50257
D = 768
B = 4
S = 2048
N = S * B
NC = 2
NS = 16
NW = NC * NS
RPW = N // NW
SPW = S // NW
NCH = 8
C = RPW // NCH
CS = C // B
LANES = 16
NDB = D // LANES
NBUF = 4
LOOK = 2


def _sc_embed(idx3, wte, wpe):
    mesh = plsc.VectorSubcoreMesh(core_axis_name="c", subcore_axis_name="s")

    @functools.partial(
        pl.kernel,
        mesh=mesh,
        out_type=jax.ShapeDtypeStruct((S, B, D), jnp.float32),
        scratch_types=[
            pltpu.VMEM((NCH, C), jnp.int32),
            pltpu.VMEM((C, D), jnp.float32),
            pltpu.VMEM((C, D), jnp.float32),
            pltpu.VMEM((C, D), jnp.float32),
            pltpu.VMEM((C, D), jnp.float32),
            pltpu.VMEM((CS, D), jnp.float32),
            pltpu.VMEM((CS, D), jnp.float32),
            pltpu.SemaphoreType.DMA,
            pltpu.SemaphoreType.DMA,
            pltpu.SemaphoreType.DMA,
            pltpu.SemaphoreType.DMA,
            pltpu.SemaphoreType.DMA,
            pltpu.SemaphoreType.DMA,
            pltpu.SemaphoreType.DMA,
            pltpu.SemaphoreType.DMA,
            pltpu.SemaphoreType.DMA,
            pltpu.SemaphoreType.DMA,
        ],
    )
    def k(idx_hbm, wte_hbm, wpe_hbm, out_hbm,
          idx_v, r0, r1, r2, r3, p0, p1,
          g0, g1, g2, g3, w0, w1, w2, w3, q0, q1):
        bufs = (r0, r1, r2, r3)
        pes = (p0, p1)
        gsems = (g0, g1, g2, g3)
        wsems = (w0, w1, w2, w3)
        qsems = (q0, q1)
        wid = lax.axis_index("s") * NC + lax.axis_index("c")
        s0 = wid * SPW
        pltpu.sync_copy(idx_hbm.at[wid], idx_v)
        gdescs = [None] * NBUF
        wdescs = [None] * NBUF
        pdescs = [None] * 2
        for j in range(2):
            pdescs[j] = pltpu.async_copy(
                wpe_hbm.at[pl.ds(s0 + j * CS, CS)], pes[j], qsems[j])
        for j in range(LOOK):
            gdescs[j] = pltpu.async_copy(
                wte_hbm.at[idx_v.at[j]], bufs[j], gsems[j])
        for j in range(NCH):
            b = j % NBUF
            jl = j + LOOK
            if jl < NCH:
                nb = jl % NBUF
                if j >= LOOK:
                    for d in wdescs[nb]:
                        d.wait()
                gdescs[nb] = pltpu.async_copy(
                    wte_hbm.at[idx_v.at[jl]], bufs[nb], gsems[nb])
            if j + 1 < NCH and j >= 1:
                pdescs[(j + 1) % 2] = pltpu.async_copy(
                    wpe_hbm.at[pl.ds(s0 + (j + 1) * CS, CS)],
                    pes[(j + 1) % 2], qsems[(j + 1) % 2])
            gdescs[b].wait()
            pdescs[j % 2].wait()
            buf = bufs[b]
            pe = pes[j % 2]

            def add_body(i, carry, buf=buf, pe=pe):
                off = i * LANES
                for sl in range(CS):
                    w = pe[sl, pl.ds(off, LANES)]
                    for bb in range(B):
                        r = sl * B + bb
                        buf[r, pl.ds(off, LANES)] = (
                            buf[r, pl.ds(off, LANES)] + w
                        )
                return carry

            lax.fori_loop(0, NDB, add_body, 0)
            ds = []
            for sl in range(CS):
                ds.append(pltpu.async_copy(
                    buf.at[pl.ds(sl * B, B)],
                    out_hbm.at[s0 + j * CS + sl],
                    wsems[b]))
            wdescs[b] = ds
        for j in range(NCH - NBUF, NCH):
            if wdescs[j % NBUF] is not None:
                for d in wdescs[j % NBUF]:
                    d.wait()

    return k(idx3, wte, wpe)


def kernel(input_ids, wte, wpe):
    idx3 = jnp.transpose(input_ids).reshape(NW, NCH, C)
    hidden = _sc_embed(idx3, wte, wpe)
    return (hidden, input_ids)
